# baseline (speedup 1.0000x reference)
import numpy as np
import jax
import jax.numpy as jnp
from jax import lax
from jax.experimental import pallas as pl
from jax.experimental.pallas import tpu as pltpu

_N = 4096
_BOX = 20.0
_CUTOFF = 2.5
_ROWS = 128
_COLT = 128
_NRB = _N // _ROWS
_NCT = _N // _COLT

_DEG = 8
_NODES = 32

_theta = np.pi * (np.arange(_NODES) + 0.5) / _NODES
_UNODES = np.cos(_theta)
_SNODES = np.sqrt((_UNODES + 1.0) / 2.0)
_DPLUS = (0.5 * _CUTOFF) * (_SNODES + 1.0)
_DMINUS = (0.5 * _CUTOFF) * (1.0 - _SNODES)
_DCT = (2.0 / _NODES) * np.cos(np.outer(np.arange(_DEG), _theta))
_DCT[0] *= 0.5


def _energy_kernel(cnt_ref, tiles_ref, xyz_ref, xt_ref, c_ref, out_ref):
    row_l = lax.broadcasted_iota(jnp.int32, (_ROWS, _COLT), 0)
    col_l = lax.broadcasted_iota(jnp.int32, (_ROWS, _COLT), 1)
    tri = col_l > row_l

    def row_body(i, total):
        rows = [xyz_ref[pl.ds(i * _ROWS, _ROWS), c : c + 1] for c in range(3)]

        def tile_sum(ct, extra_mask):
            c0 = ct * _COLT
            dsq = jnp.zeros((_ROWS, _COLT), jnp.float32)
            for c in range(3):
                col = xt_ref[c : c + 1, pl.ds(c0, _COLT)]
                d = col - rows[c]
                d = d - _BOX * jnp.round(d * (1.0 / _BOX))
                dsq = dsq + d * d

            mask = (dsq < _CUTOFF * _CUTOFF) & (dsq > 0.0)
            if extra_mask is not None:
                mask = mask & extra_mask
            dist = jnp.sqrt(jnp.where(mask, dsq, 1.0))

            s = (2.0 / _CUTOFF) * dist - 1.0
            u2 = 4.0 * s * s - 2.0
            pa = jnp.zeros((_ROWS, _COLT), jnp.float32)
            pb = jnp.zeros((_ROWS, _COLT), jnp.float32)
            qa = jnp.zeros((_ROWS, _COLT), jnp.float32)
            qb = jnp.zeros((_ROWS, _COLT), jnp.float32)
            for k in range(_DEG - 1, 0, -1):
                pa, pb = c_ref[k] + u2 * pa - pb, pa
                qa, qb = c_ref[_DEG + k] + u2 * qa - qb, qa
            p = c_ref[0] + 0.5 * u2 * pa - pb
            q = c_ref[_DEG] + 0.5 * u2 * qa - qb
            e = p + s * q

            return jnp.sum(jnp.where(mask, e, 0.0))

        def tile_body(t, acc):
            return acc + tile_sum(tiles_ref[i, t], None)

        total = total + tile_sum(i, tri)
        return lax.fori_loop(0, cnt_ref[i], tile_body, total)

    total = lax.fori_loop(0, _NRB, row_body, jnp.float32(0.0))
    out_ref[0, 0] = total + total


def kernel(xyz, W1, b1, W2, b2):
    dp = jnp.asarray(_DPLUS, jnp.float32)
    dm = jnp.asarray(_DMINUS, jnp.float32)
    sn = jnp.asarray(_SNODES, jnp.float32)
    ap = dp[:, None] @ W1 + b1
    am = dm[:, None] @ W1 + b1
    fp = (jnp.tanh(ap) @ W2)[:, 0] + b2[0]
    fm = (jnp.tanh(am) @ W2)[:, 0] + b2[0]
    even = 0.5 * (fp + fm)
    z = _CUTOFF * (sn[:, None] * W1[0])
    zsafe = jnp.where(jnp.abs(z) < 1e-4, 1.0, z)
    sinhc = jnp.where(jnp.abs(z) < 1e-4, 1.0, jnp.sinh(z) / zsafe)
    ratio = (0.5 * _CUTOFF) * W1[0] * sinhc / (jnp.cosh(ap) * jnp.cosh(am))
    odd = ratio @ W2[:, 0]
    dct = jnp.asarray(_DCT, jnp.float32)
    coef = jnp.concatenate([dct @ even, dct @ odd])

    order = jnp.argsort(xyz[:, 0])
    xyzs = xyz[order]
    xs = xyzs[:, 0]

    xb = xs.reshape(_NRB, _ROWS)
    lo_val = xb[:, 0] - _CUTOFF
    hi_val = xb[:, -1] + _CUTOFF
    full = (hi_val - lo_val) >= _BOX
    lo_m = jnp.mod(lo_val, _BOX)
    hi_m = jnp.mod(hi_val, _BOX)
    lo_idx = jnp.searchsorted(xs, lo_m, side="left").astype(jnp.int32)
    hi_idx = jnp.searchsorted(xs, hi_m, side="right").astype(jnp.int32)
    start_tile = lo_idx // _COLT
    end_tile = (hi_idx + _COLT - 1) // _COLT
    n_lin = end_tile - start_tile
    n_wrap = _NCT - start_tile + end_tile
    n_tiles = jnp.where(hi_m >= lo_m, n_lin, n_wrap)
    n_tiles = jnp.where(full, _NCT, n_tiles)
    n_tiles = jnp.clip(n_tiles, 1, _NCT).astype(jnp.int32)
    start_tile = start_tile.astype(jnp.int32)

    ct = jnp.arange(_NCT, dtype=jnp.int32)[None, :]
    in_win = ((ct - start_tile[:, None]) % _NCT) < n_tiles[:, None]
    keep = in_win & (ct > jnp.arange(_NRB, dtype=jnp.int32)[:, None])
    cnt = keep.sum(axis=1).astype(jnp.int32)
    pack = jnp.argsort(~keep, axis=1, stable=True)
    tiles = jnp.take_along_axis(jnp.broadcast_to(ct, (_NRB, _NCT)), pack,
                                axis=1).astype(jnp.int32)

    out = pl.pallas_call(
        _energy_kernel,
        in_specs=[
            pl.BlockSpec(memory_space=pltpu.SMEM),
            pl.BlockSpec(memory_space=pltpu.SMEM),
            pl.BlockSpec((_N, 3), lambda: (0, 0)),
            pl.BlockSpec((3, _N), lambda: (0, 0)),
            pl.BlockSpec(memory_space=pltpu.SMEM),
        ],
        out_specs=pl.BlockSpec(memory_space=pltpu.SMEM),
        out_shape=jax.ShapeDtypeStruct((1, 1), jnp.float32),
    )(cnt, tiles, xyzs, xyzs.T, coef)
    return out[0, 0]


# speedup vs baseline: 1.0225x; 1.0225x over previous
import numpy as np
import jax
import jax.numpy as jnp
from jax import lax
from jax.experimental import pallas as pl
from jax.experimental.pallas import tpu as pltpu

_N = 4096
_BOX = 20.0
_CUTOFF = 2.5
_ROWS = 256
_COLT = 256
_NRB = _N // _ROWS
_NCT = _N // _COLT

_DEG = 8
_NODES = 32

_theta = np.pi * (np.arange(_NODES) + 0.5) / _NODES
_UNODES = np.cos(_theta)
_SNODES = np.sqrt((_UNODES + 1.0) / 2.0)
_DPLUS = (0.5 * _CUTOFF) * (_SNODES + 1.0)
_DMINUS = (0.5 * _CUTOFF) * (1.0 - _SNODES)
_DCT = (2.0 / _NODES) * np.cos(np.outer(np.arange(_DEG), _theta))
_DCT[0] *= 0.5


def _energy_kernel(cnt_ref, tiles_ref, xyz_ref, xt_ref, c_ref, out_ref):
    row_l = lax.broadcasted_iota(jnp.int32, (_ROWS, _COLT), 0)
    col_l = lax.broadcasted_iota(jnp.int32, (_ROWS, _COLT), 1)
    tri = col_l > row_l

    def row_body(i, total):
        rows = [xyz_ref[pl.ds(i * _ROWS, _ROWS), c : c + 1] for c in range(3)]

        def tile_sum(ct, extra_mask):
            c0 = ct * _COLT
            dsq = jnp.zeros((_ROWS, _COLT), jnp.float32)
            for c in range(3):
                col = xt_ref[c : c + 1, pl.ds(c0, _COLT)]
                d = col - rows[c]
                d = d - _BOX * jnp.round(d * (1.0 / _BOX))
                dsq = dsq + d * d

            mask = (dsq < _CUTOFF * _CUTOFF) & (dsq > 0.0)
            if extra_mask is not None:
                mask = mask & extra_mask
            dist = jnp.sqrt(jnp.where(mask, dsq, 1.0))

            s = (2.0 / _CUTOFF) * dist - 1.0
            u2 = 4.0 * s * s - 2.0
            pa = jnp.zeros((_ROWS, _COLT), jnp.float32)
            pb = jnp.zeros((_ROWS, _COLT), jnp.float32)
            qa = jnp.zeros((_ROWS, _COLT), jnp.float32)
            qb = jnp.zeros((_ROWS, _COLT), jnp.float32)
            for k in range(_DEG - 1, 0, -1):
                pa, pb = c_ref[k] + u2 * pa - pb, pa
                qa, qb = c_ref[_DEG + k] + u2 * qa - qb, qa
            p = c_ref[0] + 0.5 * u2 * pa - pb
            q = c_ref[_DEG] + 0.5 * u2 * qa - qb
            e = p + s * q

            return jnp.sum(jnp.where(mask, e, 0.0))

        def tile_body(t, acc):
            return acc + tile_sum(tiles_ref[i, t], None)

        total = total + tile_sum(i, tri)
        return lax.fori_loop(0, cnt_ref[i], tile_body, total)

    total = lax.fori_loop(0, _NRB, row_body, jnp.float32(0.0))
    out_ref[0, 0] = total + total


def kernel(xyz, W1, b1, W2, b2):
    dp = jnp.asarray(_DPLUS, jnp.float32)
    dm = jnp.asarray(_DMINUS, jnp.float32)
    sn = jnp.asarray(_SNODES, jnp.float32)
    ap = dp[:, None] @ W1 + b1
    am = dm[:, None] @ W1 + b1
    fp = (jnp.tanh(ap) @ W2)[:, 0] + b2[0]
    fm = (jnp.tanh(am) @ W2)[:, 0] + b2[0]
    even = 0.5 * (fp + fm)
    z = _CUTOFF * (sn[:, None] * W1[0])
    zsafe = jnp.where(jnp.abs(z) < 1e-4, 1.0, z)
    sinhc = jnp.where(jnp.abs(z) < 1e-4, 1.0, jnp.sinh(z) / zsafe)
    ratio = (0.5 * _CUTOFF) * W1[0] * sinhc / (jnp.cosh(ap) * jnp.cosh(am))
    odd = ratio @ W2[:, 0]
    dct = jnp.asarray(_DCT, jnp.float32)
    coef = jnp.concatenate([dct @ even, dct @ odd])

    order = jnp.argsort(xyz[:, 0])
    xyzs = xyz[order]
    xs = xyzs[:, 0]

    xb = xs.reshape(_NRB, _ROWS)
    lo_val = xb[:, 0] - _CUTOFF
    hi_val = xb[:, -1] + _CUTOFF
    full = (hi_val - lo_val) >= _BOX
    lo_m = jnp.mod(lo_val, _BOX)
    hi_m = jnp.mod(hi_val, _BOX)
    lo_idx = jnp.searchsorted(xs, lo_m, side="left").astype(jnp.int32)
    hi_idx = jnp.searchsorted(xs, hi_m, side="right").astype(jnp.int32)
    start_tile = lo_idx // _COLT
    end_tile = (hi_idx + _COLT - 1) // _COLT
    n_lin = end_tile - start_tile
    n_wrap = _NCT - start_tile + end_tile
    n_tiles = jnp.where(hi_m >= lo_m, n_lin, n_wrap)
    n_tiles = jnp.where(full, _NCT, n_tiles)
    n_tiles = jnp.clip(n_tiles, 1, _NCT).astype(jnp.int32)
    start_tile = start_tile.astype(jnp.int32)

    ct = jnp.arange(_NCT, dtype=jnp.int32)[None, :]
    in_win = ((ct - start_tile[:, None]) % _NCT) < n_tiles[:, None]
    keep = in_win & (ct > jnp.arange(_NRB, dtype=jnp.int32)[:, None])
    cnt = keep.sum(axis=1).astype(jnp.int32)
    pack = jnp.argsort(~keep, axis=1, stable=True)
    tiles = jnp.take_along_axis(jnp.broadcast_to(ct, (_NRB, _NCT)), pack,
                                axis=1).astype(jnp.int32)

    out = pl.pallas_call(
        _energy_kernel,
        in_specs=[
            pl.BlockSpec(memory_space=pltpu.SMEM),
            pl.BlockSpec(memory_space=pltpu.SMEM),
            pl.BlockSpec((_N, 3), lambda: (0, 0)),
            pl.BlockSpec((3, _N), lambda: (0, 0)),
            pl.BlockSpec(memory_space=pltpu.SMEM),
        ],
        out_specs=pl.BlockSpec(memory_space=pltpu.SMEM),
        out_shape=jax.ShapeDtypeStruct((1, 1), jnp.float32),
    )(cnt, tiles, xyzs, xyzs.T, coef)
    return out[0, 0]


# speedup vs baseline: 1.0530x; 1.0298x over previous
import numpy as np
import jax
import jax.numpy as jnp
from jax import lax
from jax.experimental import pallas as pl
from jax.experimental.pallas import tpu as pltpu

_N = 4096
_BOX = 20.0
_CUTOFF = 2.5
_ROWS = 256
_COLT = 256
_NRB = _N // _ROWS
_NCT = _N // _COLT

_DEG = 8
_NODES = 32

_theta = np.pi * (np.arange(_NODES) + 0.5) / _NODES
_UNODES = np.cos(_theta)
_SNODES = np.sqrt((_UNODES + 1.0) / 2.0)
_DPLUS = (0.5 * _CUTOFF) * (_SNODES + 1.0)
_DMINUS = (0.5 * _CUTOFF) * (1.0 - _SNODES)
_DCT = (2.0 / _NODES) * np.cos(np.outer(np.arange(_DEG), _theta))
_DCT[0] *= 0.5


def _energy_kernel(cnt_ref, tiles_ref, xyz_ref, xt_ref, c_ref, out_ref):
    row_l = lax.broadcasted_iota(jnp.int32, (_ROWS, _COLT), 0)
    col_l = lax.broadcasted_iota(jnp.int32, (_ROWS, _COLT), 1)
    tri = col_l > row_l

    def row_body(i, total):
        rows = [xyz_ref[pl.ds(i * _ROWS, _ROWS), c : c + 1] for c in range(3)]

        def tile_sum(ct, extra_mask):
            c0 = ct * _COLT
            dsq = jnp.zeros((_ROWS, _COLT), jnp.float32)
            for c in range(3):
                col = xt_ref[c : c + 1, pl.ds(c0, _COLT)]
                d = col - rows[c]
                d = d - _BOX * jnp.round(d * (1.0 / _BOX))
                dsq = dsq + d * d

            mask = (dsq < _CUTOFF * _CUTOFF) & (dsq > 0.0)
            if extra_mask is not None:
                mask = mask & extra_mask
            dist = jnp.sqrt(dsq)

            s = (2.0 / _CUTOFF) * dist - 1.0
            u2 = 4.0 * s * s - 2.0
            pa = jnp.zeros((_ROWS, _COLT), jnp.float32)
            pb = jnp.zeros((_ROWS, _COLT), jnp.float32)
            qa = jnp.zeros((_ROWS, _COLT), jnp.float32)
            qb = jnp.zeros((_ROWS, _COLT), jnp.float32)
            for k in range(_DEG - 1, 0, -1):
                pa, pb = c_ref[k] + u2 * pa - pb, pa
                qa, qb = c_ref[_DEG + k] + u2 * qa - qb, qa
            p = c_ref[0] + 0.5 * u2 * pa - pb
            q = c_ref[_DEG] + 0.5 * u2 * qa - qb
            e = p + s * q

            return jnp.sum(jnp.where(mask, e, 0.0))

        def tile_body(t, acc):
            return acc + tile_sum(tiles_ref[i, t], None)

        total = total + tile_sum(i, tri)
        return lax.fori_loop(0, cnt_ref[i], tile_body, total)

    total = lax.fori_loop(0, _NRB, row_body, jnp.float32(0.0))
    out_ref[0, 0] = total + total


def kernel(xyz, W1, b1, W2, b2):
    dp = jnp.asarray(_DPLUS, jnp.float32)
    dm = jnp.asarray(_DMINUS, jnp.float32)
    sn = jnp.asarray(_SNODES, jnp.float32)
    ap = dp[:, None] @ W1 + b1
    am = dm[:, None] @ W1 + b1
    fp = (jnp.tanh(ap) @ W2)[:, 0] + b2[0]
    fm = (jnp.tanh(am) @ W2)[:, 0] + b2[0]
    even = 0.5 * (fp + fm)
    z = _CUTOFF * (sn[:, None] * W1[0])
    zsafe = jnp.where(jnp.abs(z) < 1e-4, 1.0, z)
    sinhc = jnp.where(jnp.abs(z) < 1e-4, 1.0, jnp.sinh(z) / zsafe)
    ratio = (0.5 * _CUTOFF) * W1[0] * sinhc / (jnp.cosh(ap) * jnp.cosh(am))
    odd = ratio @ W2[:, 0]
    dct = jnp.asarray(_DCT, jnp.float32)
    coef = jnp.concatenate([dct @ even, dct @ odd])

    order = jnp.argsort(xyz[:, 0])
    xyzs = xyz[order]
    xs = xyzs[:, 0]

    xb = xs.reshape(_NRB, _ROWS)
    lo_val = xb[:, 0] - _CUTOFF
    hi_val = xb[:, -1] + _CUTOFF
    full = (hi_val - lo_val) >= _BOX
    lo_m = jnp.mod(lo_val, _BOX)
    hi_m = jnp.mod(hi_val, _BOX)
    lo_idx = jnp.searchsorted(xs, lo_m, side="left").astype(jnp.int32)
    hi_idx = jnp.searchsorted(xs, hi_m, side="right").astype(jnp.int32)
    start_tile = lo_idx // _COLT
    end_tile = (hi_idx + _COLT - 1) // _COLT
    n_lin = end_tile - start_tile
    n_wrap = _NCT - start_tile + end_tile
    n_tiles = jnp.where(hi_m >= lo_m, n_lin, n_wrap)
    n_tiles = jnp.where(full, _NCT, n_tiles)
    n_tiles = jnp.clip(n_tiles, 1, _NCT).astype(jnp.int32)
    start_tile = start_tile.astype(jnp.int32)

    ct = jnp.arange(_NCT, dtype=jnp.int32)[None, :]
    in_win = ((ct - start_tile[:, None]) % _NCT) < n_tiles[:, None]
    keep = in_win & (ct > jnp.arange(_NRB, dtype=jnp.int32)[:, None])
    cnt = keep.sum(axis=1).astype(jnp.int32)
    pack = jnp.argsort(~keep, axis=1, stable=True)
    tiles = jnp.take_along_axis(jnp.broadcast_to(ct, (_NRB, _NCT)), pack,
                                axis=1).astype(jnp.int32)

    out = pl.pallas_call(
        _energy_kernel,
        in_specs=[
            pl.BlockSpec(memory_space=pltpu.SMEM),
            pl.BlockSpec(memory_space=pltpu.SMEM),
            pl.BlockSpec((_N, 3), lambda: (0, 0)),
            pl.BlockSpec((3, _N), lambda: (0, 0)),
            pl.BlockSpec(memory_space=pltpu.SMEM),
        ],
        out_specs=pl.BlockSpec(memory_space=pltpu.SMEM),
        out_shape=jax.ShapeDtypeStruct((1, 1), jnp.float32),
    )(cnt, tiles, xyzs, xyzs.T, coef)
    return out[0, 0]
